# baseline (speedup 1.0000x reference)
import functools

import jax
import jax.numpy as jnp
from jax import lax
from jax.experimental import pallas as pl
from jax.experimental.pallas import tpu as pltpu
from jax.experimental.pallas import tpu_sc as plsc

D = 256
L = 16
DC2 = D // 32
NW = 32
CHUNK = 128
GPC = CHUNK // L


def _make_sc_kernel(e_pad: int):
    epw = e_pad // NW
    nchunk = epw // CHUNK
    mesh = plsc.VectorSubcoreMesh(core_axis_name="c", subcore_axis_name="s")

    @functools.partial(
        pl.kernel,
        mesh=mesh,
        out_type=jax.ShapeDtypeStruct((e_pad,), jnp.float32),
        compiler_params=pltpu.CompilerParams(needs_layout_passes=False),
        scratch_types=[
            pltpu.VMEM((nchunk, CHUNK), jnp.int32),
            pltpu.VMEM((nchunk, CHUNK), jnp.int32),
            pltpu.VMEM((CHUNK, D // 2), jnp.int32),
            pltpu.VMEM((CHUNK, D // 2), jnp.int32),
            pltpu.VMEM((epw,), jnp.float32),
            pltpu.VMEM((D,), jnp.float32),
            pltpu.VMEM((L,), jnp.float32),
            pltpu.SemaphoreType.DMA,
            pltpu.SemaphoreType.DMA,
        ],
    )
    def sc_kernel(h_hbm, src_hbm, dst_hbm, w_hbm, b_hbm, out_hbm,
                  sidx_v, didx_v, u_buf, v_buf, out_v, w_v, b_v,
                  sem_u, sem_v):
        wid = lax.axis_index("s") * 2 + lax.axis_index("c")
        pltpu.sync_copy(src_hbm.at[wid], sidx_v)
        pltpu.sync_copy(dst_hbm.at[wid], didx_v)
        pltpu.sync_copy(w_hbm, w_v)
        pltpu.sync_copy(b_hbm, b_v)
        w_regs = [w_v[pl.ds(j * L, L)] for j in range(2 * DC2)]
        b_reg = b_v[...]
        iota = lax.iota(jnp.int32, L)
        lane_masks = [iota == e for e in range(L)]

        def chunk_body(k, carry):
            cp_u = pltpu.async_copy(
                h_hbm.at[sidx_v.at[k]], u_buf, sem_u)
            cp_v = pltpu.async_copy(
                h_hbm.at[didx_v.at[k]], v_buf, sem_v)
            cp_u.wait()
            cp_v.wait()

            def group_body(g, carry2):
                e0 = g * L
                accs = [None] * L
                for c in range(DC2):
                    w0 = w_regs[2 * c]
                    w1 = w_regs[2 * c + 1]
                    for e in range(L):
                        u32 = plsc.bitcast(
                            u_buf[e0 + e, pl.ds(c * L, L)], jnp.bfloat16)
                        v32 = plsc.bitcast(
                            v_buf[e0 + e, pl.ds(c * L, L)], jnp.bfloat16)
                        u0, u1 = plsc.unpack(
                            u32, format=plsc.PackFormat.INTERLEAVED)
                        v0, v1 = plsc.unpack(
                            v32, format=plsc.PackFormat.INTERLEAVED)
                        p = u0 * (v0 * w0) + u1 * (v1 * w1)
                        accs[e] = p if c == 0 else accs[e] + p
                tot = b_reg
                for e in range(L):
                    s = jnp.sum(accs[e])
                    tot = jnp.where(lane_masks[e],
                                    jnp.broadcast_to(s, (L,)), tot)
                out_v[pl.ds(k * CHUNK + e0, L)] = tot + b_reg
                return carry2

            lax.fori_loop(0, GPC, group_body, 0)
            return carry

        lax.fori_loop(0, nchunk, chunk_body, 0)
        pltpu.sync_copy(out_v, out_hbm.at[pl.ds(wid * epw, epw)])

    return sc_kernel


def kernel(h, edge_index, W1_w, W1_b):
    e = edge_index.shape[1]
    e_pad = ((e + NW * CHUNK - 1) // (NW * CHUNK)) * (NW * CHUNK)
    src = edge_index[0].astype(jnp.int32)
    dst = edge_index[1].astype(jnp.int32)
    pad = e_pad - e
    if pad:
        src = jnp.concatenate([src, jnp.zeros((pad,), jnp.int32)])
        dst = jnp.concatenate([dst, jnp.zeros((pad,), jnp.int32)])
    epw = e_pad // NW
    src = src.reshape(NW, epw // CHUNK, CHUNK)
    dst = dst.reshape(NW, epw // CHUNK, CHUNK)
    w = W1_w.reshape(D).astype(jnp.float32)
    w = w.reshape(DC2, L, 2).transpose(0, 2, 1).reshape(D)
    bvec = jnp.broadcast_to(W1_b.reshape(1).astype(jnp.float32), (L,))
    n = h.shape[0]
    h32 = lax.bitcast_convert_type(
        h.astype(jnp.bfloat16).reshape(n, D // 2, 2), jnp.int32)
    out = _make_sc_kernel(e_pad)(h32, src, dst, w, bvec)
    return out[:e]


# speedup vs baseline: 1.0668x; 1.0668x over previous
import functools

import jax
import jax.numpy as jnp
from jax import lax
from jax.experimental import pallas as pl
from jax.experimental.pallas import tpu as pltpu
from jax.experimental.pallas import tpu_sc as plsc

D = 256
L = 16
DC2 = D // 32
NC = 2
NS = 16
NW = NC * NS
CHUNK = 128
GPC = CHUNK // L
PAIR_CHUNKS = 80
F_CHUNKS = 56
S_CHUNKS = PAIR_CHUNKS - F_CHUNKS
FAST_CORE = 1


def _make_sc_kernel(e_pad: int):
    assert e_pad == NS * PAIR_CHUNKS * CHUNK
    mesh = plsc.VectorSubcoreMesh(core_axis_name="c", subcore_axis_name="s")

    @functools.partial(
        pl.kernel,
        mesh=mesh,
        out_type=jax.ShapeDtypeStruct((e_pad,), jnp.float32),
        compiler_params=pltpu.CompilerParams(needs_layout_passes=False),
        scratch_types=[
            pltpu.VMEM((F_CHUNKS, CHUNK), jnp.int32),
            pltpu.VMEM((F_CHUNKS, CHUNK), jnp.int32),
            pltpu.VMEM((CHUNK, D // 2), jnp.int32),
            pltpu.VMEM((CHUNK, D // 2), jnp.int32),
            pltpu.VMEM((F_CHUNKS * CHUNK,), jnp.float32),
            pltpu.VMEM((D,), jnp.float32),
            pltpu.VMEM((L,), jnp.float32),
            pltpu.SemaphoreType.DMA,
            pltpu.SemaphoreType.DMA,
        ],
    )
    def sc_kernel(h_hbm, src_hbm, dst_hbm, w_hbm, b_hbm, out_hbm,
                  sidx_v, didx_v, u_buf, v_buf, out_v, w_v, b_v,
                  sem_u, sem_v):
        cidx = lax.axis_index("c")
        sidx = lax.axis_index("s")
        pltpu.sync_copy(w_hbm, w_v)
        pltpu.sync_copy(b_hbm, b_v)
        w_regs = [w_v[pl.ds(j * L, L)] for j in range(2 * DC2)]
        b_reg = b_v[...]
        iota = lax.iota(jnp.int32, L)
        lane_masks = [iota == e for e in range(L)]

        def run(nch, chunk_base):
            pltpu.sync_copy(src_hbm.at[pl.ds(chunk_base, nch)],
                            sidx_v.at[pl.ds(0, nch)])
            pltpu.sync_copy(dst_hbm.at[pl.ds(chunk_base, nch)],
                            didx_v.at[pl.ds(0, nch)])

            def chunk_body(k, carry):
                cp_u = pltpu.async_copy(
                    h_hbm.at[sidx_v.at[k]], u_buf, sem_u)
                cp_v = pltpu.async_copy(
                    h_hbm.at[didx_v.at[k]], v_buf, sem_v)
                cp_u.wait()
                cp_v.wait()

                def group_body(g, carry2):
                    e0 = g * L
                    accs = [None] * L
                    for c in range(DC2):
                        w0 = w_regs[2 * c]
                        w1 = w_regs[2 * c + 1]
                        for e in range(L):
                            u32 = plsc.bitcast(
                                u_buf[e0 + e, pl.ds(c * L, L)], jnp.bfloat16)
                            v32 = plsc.bitcast(
                                v_buf[e0 + e, pl.ds(c * L, L)], jnp.bfloat16)
                            u0, u1 = plsc.unpack(
                                u32, format=plsc.PackFormat.INTERLEAVED)
                            v0, v1 = plsc.unpack(
                                v32, format=plsc.PackFormat.INTERLEAVED)
                            p = u0 * (v0 * w0) + u1 * (v1 * w1)
                            accs[e] = p if c == 0 else accs[e] + p
                    tot = b_reg
                    for e in range(L):
                        s = jnp.sum(accs[e])
                        tot = jnp.where(lane_masks[e],
                                        jnp.broadcast_to(s, (L,)), tot)
                    out_v[pl.ds(k * CHUNK + e0, L)] = tot + b_reg
                    return carry2

                lax.fori_loop(0, GPC, group_body, 0)
                return carry

            lax.fori_loop(0, nch, chunk_body, 0)
            pltpu.sync_copy(
                out_v.at[pl.ds(0, nch * CHUNK)],
                out_hbm.at[pl.ds(chunk_base * CHUNK, nch * CHUNK)])

        @pl.when(cidx == FAST_CORE)
        def _():
            run(F_CHUNKS, sidx * F_CHUNKS)

        @pl.when(cidx != FAST_CORE)
        def _():
            run(S_CHUNKS, NS * F_CHUNKS + sidx * S_CHUNKS)

    return sc_kernel


def kernel(h, edge_index, W1_w, W1_b):
    e = edge_index.shape[1]
    e_pad = NS * PAIR_CHUNKS * CHUNK
    assert e <= e_pad
    src = edge_index[0].astype(jnp.int32)
    dst = edge_index[1].astype(jnp.int32)
    pad = e_pad - e
    if pad:
        src = jnp.concatenate([src, jnp.zeros((pad,), jnp.int32)])
        dst = jnp.concatenate([dst, jnp.zeros((pad,), jnp.int32)])
    src = src.reshape(e_pad // CHUNK, CHUNK)
    dst = dst.reshape(e_pad // CHUNK, CHUNK)
    w = W1_w.reshape(D).astype(jnp.float32)
    w = w.reshape(DC2, L, 2).transpose(0, 2, 1).reshape(D)
    bvec = jnp.broadcast_to(W1_b.reshape(1).astype(jnp.float32), (L,))
    n = h.shape[0]
    h32 = lax.bitcast_convert_type(
        h.astype(jnp.bfloat16).reshape(n, D // 2, 2), jnp.int32)
    out = _make_sc_kernel(e_pad)(h32, src, dst, w, bvec)
    return out[:e]


# speedup vs baseline: 1.1030x; 1.0339x over previous
import functools

import jax
import jax.numpy as jnp
from jax import lax
from jax.experimental import pallas as pl
from jax.experimental.pallas import tpu as pltpu
from jax.experimental.pallas import tpu_sc as plsc

D = 256
L = 16
DC2 = D // 32
NC = 2
NS = 16
NW = NC * NS
CHUNK = 128
GPC = CHUNK // L
PAIR_CHUNKS = 80
F_CHUNKS = 56
S_CHUNKS = PAIR_CHUNKS - F_CHUNKS
FAST_CORE = 0


def _make_sc_kernel(e_pad: int):
    assert e_pad == NS * PAIR_CHUNKS * CHUNK
    mesh = plsc.VectorSubcoreMesh(core_axis_name="c", subcore_axis_name="s")

    @functools.partial(
        pl.kernel,
        mesh=mesh,
        out_type=jax.ShapeDtypeStruct((e_pad,), jnp.float32),
        compiler_params=pltpu.CompilerParams(needs_layout_passes=False),
        scratch_types=[
            pltpu.VMEM((F_CHUNKS, CHUNK), jnp.int32),
            pltpu.VMEM((F_CHUNKS, CHUNK), jnp.int32),
            pltpu.VMEM((CHUNK, D // 2), jnp.int32),
            pltpu.VMEM((CHUNK, D // 2), jnp.int32),
            pltpu.VMEM((F_CHUNKS * CHUNK,), jnp.float32),
            pltpu.VMEM((D,), jnp.float32),
            pltpu.VMEM((L,), jnp.float32),
            pltpu.SemaphoreType.DMA,
            pltpu.SemaphoreType.DMA,
        ],
    )
    def sc_kernel(h_hbm, src_hbm, dst_hbm, w_hbm, b_hbm, out_hbm,
                  sidx_v, didx_v, u_buf, v_buf, out_v, w_v, b_v,
                  sem_u, sem_v):
        cidx = lax.axis_index("c")
        sidx = lax.axis_index("s")
        pltpu.sync_copy(w_hbm, w_v)
        pltpu.sync_copy(b_hbm, b_v)
        w_regs = [w_v[pl.ds(j * L, L)] for j in range(2 * DC2)]
        b_reg = b_v[...]
        iota = lax.iota(jnp.int32, L)
        lane_masks = [iota == e for e in range(L)]

        def run(nch, chunk_base):
            pltpu.sync_copy(src_hbm.at[pl.ds(chunk_base, nch)],
                            sidx_v.at[pl.ds(0, nch)])
            pltpu.sync_copy(dst_hbm.at[pl.ds(chunk_base, nch)],
                            didx_v.at[pl.ds(0, nch)])

            def chunk_body(k, carry):
                cp_u = pltpu.async_copy(
                    h_hbm.at[sidx_v.at[k]], u_buf, sem_u)
                cp_v = pltpu.async_copy(
                    h_hbm.at[didx_v.at[k]], v_buf, sem_v)
                cp_u.wait()
                cp_v.wait()

                def group_body(g, carry2):
                    e0 = g * L
                    accs = [None] * L
                    for c in range(DC2):
                        w0 = w_regs[2 * c]
                        w1 = w_regs[2 * c + 1]
                        for e in range(L):
                            u32 = plsc.bitcast(
                                u_buf[e0 + e, pl.ds(c * L, L)], jnp.bfloat16)
                            v32 = plsc.bitcast(
                                v_buf[e0 + e, pl.ds(c * L, L)], jnp.bfloat16)
                            u0, u1 = plsc.unpack(
                                u32, format=plsc.PackFormat.INTERLEAVED)
                            v0, v1 = plsc.unpack(
                                v32, format=plsc.PackFormat.INTERLEAVED)
                            p = u0 * (v0 * w0) + u1 * (v1 * w1)
                            accs[e] = p if c == 0 else accs[e] + p
                    tot = b_reg
                    for e in range(L):
                        s = jnp.sum(accs[e])
                        tot = jnp.where(lane_masks[e],
                                        jnp.broadcast_to(s, (L,)), tot)
                    out_v[pl.ds(k * CHUNK + e0, L)] = tot + b_reg
                    return carry2

                lax.fori_loop(0, GPC, group_body, 0)
                return carry

            lax.fori_loop(0, nch, chunk_body, 0)
            pltpu.sync_copy(
                out_v.at[pl.ds(0, nch * CHUNK)],
                out_hbm.at[pl.ds(chunk_base * CHUNK, nch * CHUNK)])

        @pl.when(cidx == FAST_CORE)
        def _():
            run(F_CHUNKS, sidx * F_CHUNKS)

        @pl.when(cidx != FAST_CORE)
        def _():
            run(S_CHUNKS, NS * F_CHUNKS + sidx * S_CHUNKS)

    return sc_kernel


def kernel(h, edge_index, W1_w, W1_b):
    e = edge_index.shape[1]
    e_pad = NS * PAIR_CHUNKS * CHUNK
    assert e <= e_pad
    src = edge_index[0].astype(jnp.int32)
    dst = edge_index[1].astype(jnp.int32)
    pad = e_pad - e
    if pad:
        src = jnp.concatenate([src, jnp.zeros((pad,), jnp.int32)])
        dst = jnp.concatenate([dst, jnp.zeros((pad,), jnp.int32)])
    src = src.reshape(e_pad // CHUNK, CHUNK)
    dst = dst.reshape(e_pad // CHUNK, CHUNK)
    w = W1_w.reshape(D).astype(jnp.float32)
    w = w.reshape(DC2, L, 2).transpose(0, 2, 1).reshape(D)
    bvec = jnp.broadcast_to(W1_b.reshape(1).astype(jnp.float32), (L,))
    n = h.shape[0]
    h32 = lax.bitcast_convert_type(
        h.astype(jnp.bfloat16).reshape(n, D // 2, 2), jnp.int32)
    out = _make_sc_kernel(e_pad)(h32, src, dst, w, bvec)
    return out[:e]


# speedup vs baseline: 2.0788x; 1.8848x over previous
import functools

import jax
import jax.numpy as jnp
from jax import lax
from jax.experimental import pallas as pl
from jax.experimental.pallas import tpu as pltpu
from jax.experimental.pallas import tpu_sc as plsc

D = 256
L = 16
DC2 = D // 32
NC = 2
NS = 16
NW = NC * NS
CHUNK = 64
GPC = CHUNK // L
PAIR_CHUNKS = 160
F_CHUNKS = 80
S_CHUNKS = PAIR_CHUNKS - F_CHUNKS
FAST_CORE = 0


def _make_sc_kernel(e_pad: int, n_nodes: int):
    assert e_pad == NS * PAIR_CHUNKS * CHUNK
    mesh = plsc.VectorSubcoreMesh(core_axis_name="c", subcore_axis_name="s")
    rows_main = (n_nodes // (8 * NS)) * 8
    rows_last = n_nodes - rows_main * (NS - 1)

    @functools.partial(
        pl.kernel,
        mesh=mesh,
        out_type=jax.ShapeDtypeStruct((e_pad,), jnp.float32),
        compiler_params=pltpu.CompilerParams(needs_layout_passes=False),
        scratch_types=[
            pltpu.VMEM((F_CHUNKS, CHUNK), jnp.int32),
            pltpu.VMEM((F_CHUNKS, CHUNK), jnp.int32),
            pltpu.VMEM((CHUNK, D // 2), jnp.int32),
            pltpu.VMEM((CHUNK, D // 2), jnp.int32),
            pltpu.VMEM((F_CHUNKS * CHUNK,), jnp.float32),
            pltpu.VMEM((D,), jnp.float32),
            pltpu.VMEM((L,), jnp.float32),
            pltpu.VMEM_SHARED((10000, D // 2), jnp.int32),
            pltpu.SemaphoreType.DMA,
            pltpu.SemaphoreType.DMA,
        ],
    )
    def sc_kernel(h_hbm, src_hbm, dst_hbm, w_hbm, b_hbm, out_hbm,
                  sidx_v, didx_v, u_buf, v_buf, out_v, w_v, b_v, h_sp,
                  sem_u, sem_v):
        cidx = lax.axis_index("c")
        sidx = lax.axis_index("s")

        @pl.when(sidx < NS - 1)
        def _():
            pltpu.sync_copy(h_hbm.at[pl.ds(sidx * rows_main, rows_main)],
                            h_sp.at[pl.ds(sidx * rows_main, rows_main)])

        @pl.when(sidx == NS - 1)
        def _():
            pltpu.sync_copy(
                h_hbm.at[pl.ds((NS - 1) * rows_main, rows_last)],
                h_sp.at[pl.ds((NS - 1) * rows_main, rows_last)])

        pltpu.sync_copy(w_hbm, w_v)
        pltpu.sync_copy(b_hbm, b_v)
        plsc.subcore_barrier()
        w_regs = [w_v[pl.ds(j * L, L)] for j in range(2 * DC2)]
        b_reg = b_v[...]
        iota = lax.iota(jnp.int32, L)
        lane_masks = [iota == e for e in range(L)]

        def run(nch, chunk_base):
            pltpu.sync_copy(src_hbm.at[pl.ds(chunk_base, nch)],
                            sidx_v.at[pl.ds(0, nch)])
            pltpu.sync_copy(dst_hbm.at[pl.ds(chunk_base, nch)],
                            didx_v.at[pl.ds(0, nch)])

            def chunk_body(k, carry):
                cp_u = pltpu.async_copy(
                    h_sp.at[sidx_v.at[k]], u_buf, sem_u)
                cp_v = pltpu.async_copy(
                    h_sp.at[didx_v.at[k]], v_buf, sem_v)
                cp_u.wait()
                cp_v.wait()

                def group_body(g, carry2):
                    e0 = g * L
                    accs = [None] * L
                    for c in range(DC2):
                        w0 = w_regs[2 * c]
                        w1 = w_regs[2 * c + 1]
                        for e in range(L):
                            u32 = plsc.bitcast(
                                u_buf[e0 + e, pl.ds(c * L, L)], jnp.bfloat16)
                            v32 = plsc.bitcast(
                                v_buf[e0 + e, pl.ds(c * L, L)], jnp.bfloat16)
                            u0, u1 = plsc.unpack(
                                u32, format=plsc.PackFormat.INTERLEAVED)
                            v0, v1 = plsc.unpack(
                                v32, format=plsc.PackFormat.INTERLEAVED)
                            p = u0 * (v0 * w0) + u1 * (v1 * w1)
                            accs[e] = p if c == 0 else accs[e] + p
                    tot = b_reg
                    for e in range(L):
                        s = jnp.sum(accs[e])
                        tot = jnp.where(lane_masks[e],
                                        jnp.broadcast_to(s, (L,)), tot)
                    out_v[pl.ds(k * CHUNK + e0, L)] = tot + b_reg
                    return carry2

                lax.fori_loop(0, GPC, group_body, 0)
                return carry

            lax.fori_loop(0, nch, chunk_body, 0)
            pltpu.sync_copy(
                out_v.at[pl.ds(0, nch * CHUNK)],
                out_hbm.at[pl.ds(chunk_base * CHUNK, nch * CHUNK)])

        @pl.when(cidx == FAST_CORE)
        def _():
            run(F_CHUNKS, sidx * F_CHUNKS)

        @pl.when(cidx != FAST_CORE)
        def _():
            run(S_CHUNKS, NS * F_CHUNKS + sidx * S_CHUNKS)

    return sc_kernel


def kernel(h, edge_index, W1_w, W1_b):
    e = edge_index.shape[1]
    e_pad = NS * PAIR_CHUNKS * CHUNK
    assert e <= e_pad
    src = edge_index[0].astype(jnp.int32)
    dst = edge_index[1].astype(jnp.int32)
    pad = e_pad - e
    if pad:
        src = jnp.concatenate([src, jnp.zeros((pad,), jnp.int32)])
        dst = jnp.concatenate([dst, jnp.zeros((pad,), jnp.int32)])
    src = src.reshape(e_pad // CHUNK, CHUNK)
    dst = dst.reshape(e_pad // CHUNK, CHUNK)
    w = W1_w.reshape(D).astype(jnp.float32)
    w = w.reshape(DC2, L, 2).transpose(0, 2, 1).reshape(D)
    bvec = jnp.broadcast_to(W1_b.reshape(1).astype(jnp.float32), (L,))
    n = h.shape[0]
    h32 = lax.bitcast_convert_type(
        h.astype(jnp.bfloat16).reshape(n, D // 2, 2), jnp.int32)
    out = _make_sc_kernel(e_pad, n)(h32, src, dst, w, bvec)
    return out[:e]


# speedup vs baseline: 2.6269x; 1.2637x over previous
import functools

import jax
import jax.numpy as jnp
from jax import lax
from jax.experimental import pallas as pl
from jax.experimental.pallas import tpu as pltpu
from jax.experimental.pallas import tpu_sc as plsc

D = 256
L = 16
DC2 = D // 32
NC = 2
NS = 16
NW = NC * NS
CHUNK = 64
GPC = CHUNK // L
PAIR_CHUNKS = 160
F_CHUNKS = 80
S_CHUNKS = PAIR_CHUNKS - F_CHUNKS
FAST_CORE = 0


def _make_sc_kernel(e_pad: int, n_nodes: int):
    assert e_pad == NS * PAIR_CHUNKS * CHUNK
    mesh = plsc.VectorSubcoreMesh(core_axis_name="c", subcore_axis_name="s")
    SB = 16
    n_blocks = n_nodes // SB
    blocks_main = n_blocks // NS
    blocks_extra = n_blocks - blocks_main * NS

    @functools.partial(
        pl.kernel,
        mesh=mesh,
        out_type=jax.ShapeDtypeStruct((e_pad,), jnp.float32),
        compiler_params=pltpu.CompilerParams(needs_layout_passes=False),
        scratch_types=[
            pltpu.VMEM((F_CHUNKS, CHUNK), jnp.int32),
            pltpu.VMEM((F_CHUNKS, CHUNK), jnp.int32),
            pltpu.VMEM((CHUNK, D // 2), jnp.int32),
            pltpu.VMEM((CHUNK, D // 2), jnp.int32),
            pltpu.VMEM((F_CHUNKS * CHUNK,), jnp.float32),
            pltpu.VMEM((D,), jnp.float32),
            pltpu.VMEM((L,), jnp.float32),
            pltpu.VMEM((SB, D), jnp.float32),
            pltpu.VMEM_SHARED((10000, D // 2), jnp.int32),
            pltpu.SemaphoreType.DMA,
            pltpu.SemaphoreType.DMA,
        ],
    )
    def sc_kernel(h_hbm, src_hbm, dst_hbm, w_hbm, b_hbm, out_hbm,
                  sidx_v, didx_v, u_buf, v_buf, out_v, w_v, b_v, stage_v,
                  h_sp, sem_u, sem_v):
        cidx = lax.axis_index("c")
        sidx = lax.axis_index("s")

        def stage_block(b, carry):
            row0 = b * SB
            pltpu.sync_copy(h_hbm.at[pl.ds(row0, SB)], stage_v)
            for r in range(SB):
                for c in range(DC2):
                    a0 = stage_v[r, pl.ds(c * 32, L)]
                    a1 = stage_v[r, pl.ds(c * 32 + L, L)]
                    packed = plsc.pack(
                        a0, a1, format=plsc.PackFormat.INTERLEAVED)
                    u_buf[r, pl.ds(c * L, L)] = plsc.bitcast(
                        packed, jnp.int32)
            pltpu.sync_copy(u_buf.at[pl.ds(0, SB)],
                            h_sp.at[pl.ds(row0, SB)])
            return carry

        tile_blocks = blocks_main + (sidx >= NS - blocks_extra)
        b0 = sidx * blocks_main + jnp.maximum(
            sidx - (NS - blocks_extra), 0) if blocks_extra else (
                sidx * blocks_main)
        lax.fori_loop(b0, b0 + tile_blocks, stage_block, 0)
        pltpu.sync_copy(w_hbm, w_v)
        pltpu.sync_copy(b_hbm, b_v)
        plsc.subcore_barrier()
        w_regs = [w_v[pl.ds(j * L, L)] for j in range(2 * DC2)]
        b_reg = b_v[...]
        iota = lax.iota(jnp.int32, L)
        lane_masks = [iota == e for e in range(L)]

        def run(nch, chunk_base):
            pltpu.sync_copy(src_hbm.at[pl.ds(chunk_base, nch)],
                            sidx_v.at[pl.ds(0, nch)])
            pltpu.sync_copy(dst_hbm.at[pl.ds(chunk_base, nch)],
                            didx_v.at[pl.ds(0, nch)])

            def chunk_body(k, carry):
                cp_u = pltpu.async_copy(
                    h_sp.at[sidx_v.at[k]], u_buf, sem_u)
                cp_v = pltpu.async_copy(
                    h_sp.at[didx_v.at[k]], v_buf, sem_v)
                cp_u.wait()
                cp_v.wait()

                def group_body(g, carry2):
                    e0 = g * L
                    accs = [None] * L
                    for c in range(DC2):
                        w0 = w_regs[2 * c]
                        w1 = w_regs[2 * c + 1]
                        for e in range(L):
                            u32 = plsc.bitcast(
                                u_buf[e0 + e, pl.ds(c * L, L)], jnp.bfloat16)
                            v32 = plsc.bitcast(
                                v_buf[e0 + e, pl.ds(c * L, L)], jnp.bfloat16)
                            u0, u1 = plsc.unpack(
                                u32, format=plsc.PackFormat.INTERLEAVED)
                            v0, v1 = plsc.unpack(
                                v32, format=plsc.PackFormat.INTERLEAVED)
                            p = u0 * (v0 * w0) + u1 * (v1 * w1)
                            accs[e] = p if c == 0 else accs[e] + p
                    tot = b_reg
                    for e in range(L):
                        s = jnp.sum(accs[e])
                        tot = jnp.where(lane_masks[e],
                                        jnp.broadcast_to(s, (L,)), tot)
                    out_v[pl.ds(k * CHUNK + e0, L)] = tot + b_reg
                    return carry2

                lax.fori_loop(0, GPC, group_body, 0)
                return carry

            lax.fori_loop(0, nch, chunk_body, 0)
            pltpu.sync_copy(
                out_v.at[pl.ds(0, nch * CHUNK)],
                out_hbm.at[pl.ds(chunk_base * CHUNK, nch * CHUNK)])

        @pl.when(cidx == FAST_CORE)
        def _():
            run(F_CHUNKS, sidx * F_CHUNKS)

        @pl.when(cidx != FAST_CORE)
        def _():
            run(S_CHUNKS, NS * F_CHUNKS + sidx * S_CHUNKS)

    return sc_kernel


def kernel(h, edge_index, W1_w, W1_b):
    e = edge_index.shape[1]
    e_pad = NS * PAIR_CHUNKS * CHUNK
    assert e <= e_pad
    src = edge_index[0].astype(jnp.int32)
    dst = edge_index[1].astype(jnp.int32)
    pad = e_pad - e
    if pad:
        src = jnp.concatenate([src, jnp.zeros((pad,), jnp.int32)])
        dst = jnp.concatenate([dst, jnp.zeros((pad,), jnp.int32)])
    src = src.reshape(e_pad // CHUNK, CHUNK)
    dst = dst.reshape(e_pad // CHUNK, CHUNK)
    w = W1_w.reshape(D).astype(jnp.float32)
    bvec = jnp.broadcast_to(W1_b.reshape(1).astype(jnp.float32), (L,))
    n = h.shape[0]
    out = _make_sc_kernel(e_pad, n)(
        h.astype(jnp.float32), src, dst, w, bvec)
    return out[:e]


# speedup vs baseline: 2.8560x; 1.0872x over previous
import functools

import jax
import jax.numpy as jnp
from jax import lax
from jax.experimental import pallas as pl
from jax.experimental.pallas import tpu as pltpu
from jax.experimental.pallas import tpu_sc as plsc

D = 256
L = 16
DC2 = D // 32
NC = 2
NS = 16
NW = NC * NS
CHUNK = 64
GPC = CHUNK // L
PAIR_CHUNKS = 160
F_CHUNKS = 80
S_CHUNKS = PAIR_CHUNKS - F_CHUNKS
FAST_CORE = 0


def _make_sc_kernel(e_pad: int, n_nodes: int):
    assert e_pad == NS * PAIR_CHUNKS * CHUNK
    mesh = plsc.VectorSubcoreMesh(core_axis_name="c", subcore_axis_name="s")
    SB = 16
    n_blocks = n_nodes // SB
    blocks_main = n_blocks // NS
    blocks_extra = n_blocks - blocks_main * NS

    @functools.partial(
        pl.kernel,
        mesh=mesh,
        out_type=jax.ShapeDtypeStruct((e_pad,), jnp.float32),
        compiler_params=pltpu.CompilerParams(needs_layout_passes=False),
        scratch_types=[
            pltpu.VMEM((F_CHUNKS, CHUNK), jnp.int32),
            pltpu.VMEM((F_CHUNKS, CHUNK), jnp.int32),
            pltpu.VMEM((CHUNK, D // 2), jnp.int32),
            pltpu.VMEM((CHUNK, D // 2), jnp.int32),
            pltpu.VMEM((F_CHUNKS * CHUNK,), jnp.float32),
            pltpu.VMEM((D,), jnp.float32),
            pltpu.VMEM((L,), jnp.float32),
            pltpu.VMEM((SB, D), jnp.float32),
            pltpu.VMEM_SHARED((10000, D // 2), jnp.int32),
            pltpu.SemaphoreType.DMA,
            pltpu.SemaphoreType.DMA,
        ],
    )
    def sc_kernel(h_hbm, src_hbm, dst_hbm, w_hbm, b_hbm, out_hbm,
                  sidx_v, didx_v, u_buf, v_buf, out_v, w_v, b_v, stage_v,
                  h_sp, sem_u, sem_v):
        cidx = lax.axis_index("c")
        sidx = lax.axis_index("s")

        def stage_block(b, carry):
            row0 = b * SB
            pltpu.sync_copy(h_hbm.at[pl.ds(row0, SB)], stage_v)
            for r in range(SB):
                for c in range(DC2):
                    a0 = stage_v[r, pl.ds(c * 32, L)]
                    a1 = stage_v[r, pl.ds(c * 32 + L, L)]
                    packed = plsc.pack(
                        a0, a1, format=plsc.PackFormat.INTERLEAVED)
                    u_buf[r, pl.ds(c * L, L)] = plsc.bitcast(
                        packed, jnp.int32)
            pltpu.sync_copy(u_buf.at[pl.ds(0, SB)],
                            h_sp.at[pl.ds(row0, SB)])
            return carry

        tile_blocks = blocks_main + (sidx >= NS - blocks_extra)
        b0 = sidx * blocks_main + jnp.maximum(
            sidx - (NS - blocks_extra), 0) if blocks_extra else (
                sidx * blocks_main)
        lax.fori_loop(b0, b0 + tile_blocks, stage_block, 0)
        pltpu.sync_copy(w_hbm, w_v)
        pltpu.sync_copy(b_hbm, b_v)
        plsc.subcore_barrier()
        w_regs = [w_v[pl.ds(j * L, L)] for j in range(2 * DC2)]
        b_reg = b_v[...]
        iota = lax.iota(jnp.int32, L)
        lane_masks = [iota == e for e in range(L)]

        def run(nch, chunk_base):
            pltpu.sync_copy(src_hbm.at[pl.ds(chunk_base, nch)],
                            sidx_v.at[pl.ds(0, nch)])
            pltpu.sync_copy(dst_hbm.at[pl.ds(chunk_base, nch)],
                            didx_v.at[pl.ds(0, nch)])

            def chunk_body(k, carry):
                cp_u = pltpu.async_copy(
                    h_sp.at[sidx_v.at[k]], u_buf, sem_u)
                cp_v = pltpu.async_copy(
                    h_sp.at[didx_v.at[k]], v_buf, sem_v)
                cp_u.wait()
                cp_v.wait()

                def group_body(g, carry2):
                    e0 = g * L
                    accs = [None] * L
                    for c in range(DC2):
                        w0 = w_regs[2 * c]
                        w1 = w_regs[2 * c + 1]
                        for e in range(L):
                            u32 = plsc.bitcast(
                                u_buf[e0 + e, pl.ds(c * L, L)], jnp.bfloat16)
                            v32 = plsc.bitcast(
                                v_buf[e0 + e, pl.ds(c * L, L)], jnp.bfloat16)
                            p32 = u32 * v32
                            p0, p1 = plsc.unpack(
                                p32, format=plsc.PackFormat.INTERLEAVED)
                            p = p0 * w0 + p1 * w1
                            accs[e] = p if c == 0 else accs[e] + p
                    tot = b_reg
                    for e in range(L):
                        s = jnp.sum(accs[e])
                        tot = jnp.where(lane_masks[e],
                                        jnp.broadcast_to(s, (L,)), tot)
                    out_v[pl.ds(k * CHUNK + e0, L)] = tot + b_reg
                    return carry2

                lax.fori_loop(0, GPC, group_body, 0)
                return carry

            lax.fori_loop(0, nch, chunk_body, 0)
            pltpu.sync_copy(
                out_v.at[pl.ds(0, nch * CHUNK)],
                out_hbm.at[pl.ds(chunk_base * CHUNK, nch * CHUNK)])

        @pl.when(cidx == FAST_CORE)
        def _():
            run(F_CHUNKS, sidx * F_CHUNKS)

        @pl.when(cidx != FAST_CORE)
        def _():
            run(S_CHUNKS, NS * F_CHUNKS + sidx * S_CHUNKS)

    return sc_kernel


def kernel(h, edge_index, W1_w, W1_b):
    e = edge_index.shape[1]
    e_pad = NS * PAIR_CHUNKS * CHUNK
    assert e <= e_pad
    src = edge_index[0].astype(jnp.int32)
    dst = edge_index[1].astype(jnp.int32)
    pad = e_pad - e
    if pad:
        src = jnp.concatenate([src, jnp.zeros((pad,), jnp.int32)])
        dst = jnp.concatenate([dst, jnp.zeros((pad,), jnp.int32)])
    src = src.reshape(e_pad // CHUNK, CHUNK)
    dst = dst.reshape(e_pad // CHUNK, CHUNK)
    w = W1_w.reshape(D).astype(jnp.float32)
    bvec = jnp.broadcast_to(W1_b.reshape(1).astype(jnp.float32), (L,))
    n = h.shape[0]
    out = _make_sc_kernel(e_pad, n)(
        h.astype(jnp.float32), src, dst, w, bvec)
    return out[:e]
